# baseline (speedup 1.0000x reference)
import functools

import jax
import jax.numpy as jnp
from jax import lax
from jax.experimental import pallas as pl
from jax.experimental.pallas import tpu as pltpu
from jax.experimental.pallas import tpu_sc as plsc

L_SEQ = 2048
N_BATCH = 4
D_MODEL = 768
LANES = 16
DV = D_MODEL // LANES

NC, NS = 2, 16
NW = NC * NS
L_PER_W = L_SEQ // NW
CH = 16
NCH = L_PER_W // CH


def _body(x_hbm, tok_hbm, pos_hbm, seg_hbm, out_hbm,
          x_v, tok_v, seg_v, c0_v, diff_v, pos_v, stage_v, sem_pos, sem_row):
    wid = lax.axis_index("s") * NC + lax.axis_index("c")
    base = wid * L_PER_W

    for n in range(N_BATCH):
        pltpu.sync_copy(x_hbm.at[n, pl.ds(base, L_PER_W)], x_v.at[n])
    pltpu.sync_copy(tok_hbm.at[pl.ds(0, 2)], tok_v)
    pltpu.sync_copy(seg_hbm.at[pl.ds(0, 2)], seg_v)

    for j in range(DV):
        sl = pl.ds(j * LANES, LANES)
        t0 = tok_v[0, sl] + seg_v[0, sl]
        t1 = tok_v[1, sl] + seg_v[1, sl]
        c0_v[sl] = t0
        diff_v[sl] = t1 - t0

    def pos_copy(c, b):
        return pltpu.make_async_copy(
            pos_hbm.at[pl.ds(base + c * CH, CH)], pos_v.at[b], sem_pos.at[b])

    def row_copy(b, i, xi, lg, n):
        return pltpu.make_async_copy(
            stage_v.at[b, i, xi], out_hbm.at[lg, n], sem_row.at[b])

    pos_copy(0, 0).start()

    def chunk_body(c, _):
        b = lax.rem(c, 2)
        nxt = lax.rem(c + 1, 2)

        @pl.when(c + 1 < NCH)
        def _():
            pos_copy(c + 1, nxt).start()

        pos_copy(c, b).wait()

        @pl.when(c >= 2)
        def _():
            def drain(k, _):
                row_copy(b, 0, 0, 0, 0).wait()
                return 0
            lax.fori_loop(0, CH * N_BATCH, drain, 0)

        xr = [x_v[n, pl.ds(c * CH, CH)] for n in range(N_BATCH)]

        for i in range(CH):
            for j in range(DV):
                sl = pl.ds(j * LANES, LANES)
                a = pos_v[b, i, sl] + c0_v[sl]
                stage_v[b, i, 0, sl] = a
                stage_v[b, i, 1, sl] = a + diff_v[sl]
            lg = base + c * CH + i
            for n in range(N_BATCH):
                row_copy(b, i, xr[n][i], lg, n).start()
        return 0

    lax.fori_loop(0, NCH, chunk_body, 0)

    def drain_tail(k, _):
        row_copy(0, 0, 0, 0, 0).wait()
        return 0
    lax.fori_loop(0, CH * N_BATCH, drain_tail, 0)

    def drain_tail2(k, _):
        row_copy(1, 0, 0, 0, 0).wait()
        return 0
    lax.fori_loop(0, CH * N_BATCH, drain_tail2, 0)


@jax.jit
def _sc_embed(x, token_table, pos2d, segment_table):
    mesh = plsc.VectorSubcoreMesh(core_axis_name="c", subcore_axis_name="s")
    kfn = pl.kernel(
        _body,
        out_type=jax.ShapeDtypeStruct((L_SEQ, N_BATCH, D_MODEL), jnp.float32),
        mesh=mesh,
        scratch_types=[
            pltpu.VMEM((N_BATCH, L_PER_W), jnp.int32),
            pltpu.VMEM((2, D_MODEL), jnp.float32),
            pltpu.VMEM((2, D_MODEL), jnp.float32),
            pltpu.VMEM((D_MODEL,), jnp.float32),
            pltpu.VMEM((D_MODEL,), jnp.float32),
            pltpu.VMEM((2, CH, D_MODEL), jnp.float32),
            pltpu.VMEM((2, CH, 2, D_MODEL), jnp.float32),
            pltpu.SemaphoreType.DMA((2,)),
            pltpu.SemaphoreType.DMA((2,)),
        ],
    )
    return kfn(x, token_table, pos2d, segment_table)


def kernel(x, token_table, pos_embedding, segment_table):
    xi = x.astype(jnp.int32)
    pos2d = pos_embedding.reshape(pos_embedding.shape[0], D_MODEL)[:L_SEQ]
    return _sc_embed(xi, token_table, pos2d, segment_table)


# speedup vs baseline: 1.3163x; 1.3163x over previous
import functools

import jax
import jax.numpy as jnp
from jax import lax
from jax.experimental import pallas as pl
from jax.experimental.pallas import tpu as pltpu
from jax.experimental.pallas import tpu_sc as plsc

L_SEQ = 2048
N_BATCH = 4
D_MODEL = 768
LANES = 16
DV = D_MODEL // LANES

NC, NS = 2, 16
NW = NC * NS
L_PER_W = L_SEQ // NW
CH = 16
NCH = L_PER_W // CH
POS_B = CH * D_MODEL
OUT_B = CH * N_BATCH * D_MODEL


def _body(x_hbm, tok_hbm, pos_hbm, seg_hbm, out_hbm,
          x_v, tok_v, seg_v, c0_v, diff_v, pos_v, out_v, sem_pos, sem_out):
    wid = lax.axis_index("s") * NC + lax.axis_index("c")
    base = wid * L_PER_W

    for n in range(N_BATCH):
        pltpu.sync_copy(x_hbm.at[n, pl.ds(base, L_PER_W)], x_v.at[n])
    pltpu.sync_copy(tok_hbm.at[pl.ds(0, 2)], tok_v)
    pltpu.sync_copy(seg_hbm.at[pl.ds(0, 2)], seg_v)

    for j in range(DV):
        sl = pl.ds(j * LANES, LANES)
        t0 = tok_v[0, sl] + seg_v[0, sl]
        t1 = tok_v[1, sl] + seg_v[1, sl]
        c0_v[sl] = t0
        diff_v[sl] = t1 - t0

    def pos_copy(c, b):
        return pltpu.make_async_copy(
            pos_hbm.at[pl.ds((base + c * CH) * D_MODEL, POS_B)],
            pos_v.at[pl.ds(b * POS_B, POS_B)], sem_pos.at[b])

    def out_copy(c, b):
        return pltpu.make_async_copy(
            out_v.at[pl.ds(b * OUT_B, OUT_B)],
            out_hbm.at[pl.ds((base + c * CH) * N_BATCH * D_MODEL, OUT_B)],
            sem_out.at[b])

    pos_copy(0, 0).start()

    def chunk_body(c, _):
        b = lax.rem(c, 2)
        nxt = lax.rem(c + 1, 2)

        @pl.when(c + 1 < NCH)
        def _():
            pos_copy(c + 1, nxt).start()

        pos_copy(c, b).wait()

        @pl.when(c >= 2)
        def _():
            out_copy(c - 2, b).wait()

        pbase = b * POS_B
        obase = b * OUT_B
        xrs = [x_v[n, pl.ds(c * CH, CH)] for n in range(N_BATCH)]

        @plsc.parallel_loop(0, CH, unroll=2)
        def i_body(i):
            spl = [xrs[n].at[jnp.full((LANES,), i, jnp.int32)]
                   .get(mode="promise_in_bounds").astype(jnp.float32)
                   for n in range(N_BATCH)]
            for j in range(DV):
                sl = pl.ds(j * LANES, LANES)
                c0j = c0_v[sl]
                dfj = diff_v[sl]
                t = pos_v[pl.ds(pbase + i * D_MODEL + j * LANES, LANES)] + c0j
                for n in range(N_BATCH):
                    off = obase + ((i * N_BATCH + n) * D_MODEL) + j * LANES
                    out_v[pl.ds(off, LANES)] = t + spl[n] * dfj

        out_copy(c, b).start()
        return 0

    lax.fori_loop(0, NCH, chunk_body, 0)

    for c in range(max(NCH - 2, 0), NCH):
        out_copy(c, c % 2).wait()


@jax.jit
def _sc_embed(x, token_table, pos2d, segment_table):
    mesh = plsc.VectorSubcoreMesh(core_axis_name="c", subcore_axis_name="s")
    kfn = pl.kernel(
        _body,
        out_type=jax.ShapeDtypeStruct((L_SEQ * N_BATCH * D_MODEL,), jnp.float32),
        mesh=mesh,
        scratch_types=[
            pltpu.VMEM((N_BATCH, L_PER_W), jnp.int32),
            pltpu.VMEM((2, D_MODEL), jnp.float32),
            pltpu.VMEM((2, D_MODEL), jnp.float32),
            pltpu.VMEM((D_MODEL,), jnp.float32),
            pltpu.VMEM((D_MODEL,), jnp.float32),
            pltpu.VMEM((2 * POS_B,), jnp.float32),
            pltpu.VMEM((2 * OUT_B,), jnp.float32),
            pltpu.SemaphoreType.DMA((2,)),
            pltpu.SemaphoreType.DMA((2,)),
        ],
    )
    out = kfn(x, token_table, pos2d, segment_table)
    return out.reshape(L_SEQ, N_BATCH, D_MODEL)


def kernel(x, token_table, pos_embedding, segment_table):
    xi = x.astype(jnp.int32)
    pos2d = pos_embedding.reshape(-1)[: L_SEQ * D_MODEL]
    return _sc_embed(xi, token_table, pos2d, segment_table)


# speedup vs baseline: 2.0944x; 1.5911x over previous
import functools

import jax
import jax.numpy as jnp
from jax import lax
from jax.experimental import pallas as pl
from jax.experimental.pallas import tpu as pltpu
from jax.experimental.pallas import tpu_sc as plsc

L_SEQ = 2048
N_BATCH = 4
D_MODEL = 768
LANES = 16
DV = D_MODEL // LANES

NC, NS = 2, 16
NW = NC * NS
L_PER_W = L_SEQ // NW
CH = 16
NCH = L_PER_W // CH
POS_B = CH * D_MODEL
OUT_B = CH * N_BATCH * D_MODEL


def _body(x_hbm, tok_hbm, pos_hbm, seg_hbm, out_hbm,
          x_v, tok_v, seg_v, c0_v, diff_v, pos_v, out_v, sem_pos, sem_out):
    wid = lax.axis_index("s") * NC + lax.axis_index("c")
    base = wid * L_PER_W

    for n in range(N_BATCH):
        pltpu.sync_copy(x_hbm.at[n, pl.ds(base, L_PER_W)], x_v.at[n])
    pltpu.sync_copy(tok_hbm.at[pl.ds(0, 2)], tok_v)
    pltpu.sync_copy(seg_hbm.at[pl.ds(0, 2)], seg_v)

    for j in range(DV):
        sl = pl.ds(j * LANES, LANES)
        t0 = tok_v[0, sl] + seg_v[0, sl]
        t1 = tok_v[1, sl] + seg_v[1, sl]
        c0_v[sl] = t0
        diff_v[sl] = t1 - t0

    def pos_copy(c, b):
        return pltpu.make_async_copy(
            pos_hbm.at[pl.ds((base + c * CH) * D_MODEL, POS_B)],
            pos_v.at[pl.ds(b * POS_B, POS_B)], sem_pos.at[b])

    def out_copy(c, b):
        return pltpu.make_async_copy(
            out_v.at[b], out_hbm.at[pl.ds(base + c * CH, CH)], sem_out.at[b])

    pos_copy(0, 0).start()

    def chunk_body(c, _):
        b = lax.rem(c, 2)
        nxt = lax.rem(c + 1, 2)

        @pl.when(c + 1 < NCH)
        def _():
            pos_copy(c + 1, nxt).start()

        pos_copy(c, b).wait()

        @pl.when(c >= 2)
        def _():
            out_copy(c - 2, b).wait()

        pbase = b * POS_B
        xrs = [x_v[n, pl.ds(c * CH, CH)] for n in range(N_BATCH)]

        @plsc.parallel_loop(0, CH, unroll=2)
        def i_body(i):
            spl = [xrs[n].at[jnp.full((LANES,), i, jnp.int32)]
                   .get(mode="promise_in_bounds").astype(jnp.float32)
                   for n in range(N_BATCH)]
            for j in range(DV):
                sl = pl.ds(j * LANES, LANES)
                c0j = c0_v[sl]
                dfj = diff_v[sl]
                t = pos_v[pl.ds(pbase + i * D_MODEL + j * LANES, LANES)] + c0j
                for n in range(N_BATCH):
                    out_v[b, i, n, sl] = t + spl[n] * dfj

        out_copy(c, b).start()
        return 0

    lax.fori_loop(0, NCH, chunk_body, 0)

    for c in range(max(NCH - 2, 0), NCH):
        out_copy(c, c % 2).wait()


@jax.jit
def _sc_embed(x, token_table, pos2d, segment_table):
    mesh = plsc.VectorSubcoreMesh(core_axis_name="c", subcore_axis_name="s")
    kfn = pl.kernel(
        _body,
        out_type=jax.ShapeDtypeStruct((L_SEQ, N_BATCH, D_MODEL), jnp.float32),
        mesh=mesh,
        scratch_types=[
            pltpu.VMEM((N_BATCH, L_PER_W), jnp.int32),
            pltpu.VMEM((2, D_MODEL), jnp.float32),
            pltpu.VMEM((2, D_MODEL), jnp.float32),
            pltpu.VMEM((D_MODEL,), jnp.float32),
            pltpu.VMEM((D_MODEL,), jnp.float32),
            pltpu.VMEM((2 * POS_B,), jnp.float32),
            pltpu.VMEM((2, CH, N_BATCH, D_MODEL), jnp.float32),
            pltpu.SemaphoreType.DMA((2,)),
            pltpu.SemaphoreType.DMA((2,)),
        ],
    )
    return kfn(x, token_table, pos2d, segment_table)


def kernel(x, token_table, pos_embedding, segment_table):
    xi = x.astype(jnp.int32)
    pos2d = pos_embedding.reshape(-1)[: L_SEQ * D_MODEL]
    return _sc_embed(xi, token_table, pos2d, segment_table)
